# baseline (speedup 1.0000x reference)
import jax
import jax.numpy as jnp
from jax.experimental import pallas as pl

_NUM_TYPES = 128
_ROWS_PER_BLOCK = 256


def _onehot_block(x_ref, out_ref):
    x = x_ref[...]
    classes = jax.lax.broadcasted_iota(jnp.int32, (1, 1, _NUM_TYPES), 2)
    out_ref[...] = (x[..., None] == classes).astype(jnp.int32)


def kernel(x):
    n, s = x.shape
    r = _ROWS_PER_BLOCK
    grid = (n // r,)
    return pl.pallas_call(
        _onehot_block,
        grid=grid,
        in_specs=[pl.BlockSpec((r, s), lambda i: (i, 0))],
        out_specs=pl.BlockSpec((r, s, _NUM_TYPES), lambda i: (i, 0, 0)),
        out_shape=jax.ShapeDtypeStruct((n, s, _NUM_TYPES), jnp.int32),
    )(x)
